# baseline (speedup 1.0000x reference)
import jax
import jax.numpy as jnp
from jax.experimental import pallas as pl
from jax.experimental.pallas import tpu as pltpu


def _leaky(v, alpha=0.2):
    return jnp.where(v >= 0, v, alpha * v)


def _bf(v):
    return v.astype(jnp.bfloat16).astype(jnp.float32)


def _fused_kernel(x_hbm, m_hbm, h_hbm, pri_hbm, inc_hbm, w_hbm, bias_hbm,
                  w2_hbm, a_hbm, a2_hbm, a3_hbm, node_ref, edge_ref,
                  vx, vm, vh, vpri, vinc, vw, vbias, vw2, va, va2, va3, sems):
    f32 = jnp.float32
    d = vw.shape[1]

    srcs = (x_hbm, m_hbm, h_hbm, w_hbm, bias_hbm, inc_hbm,
            w2_hbm, pri_hbm, a_hbm, a2_hbm, a3_hbm)
    dsts = (vx, vm, vh, vw, vbias, vinc, vw2, vpri, va, va2, va3)
    copies = [pltpu.make_async_copy(s, t, sems.at[i])
              for i, (s, t) in enumerate(zip(srcs, dsts))]
    for c in copies:
        c.start()

    for c in copies[:5]:
        c.wait()
    xcat = jnp.concatenate([vx[0], vm[0], vh[0]], axis=0)
    X = jax.lax.dot_general(xcat, vw[...], (((0,), (0,)), ((), ())),
                            preferred_element_type=f32)
    X = X + jnp.transpose(vbias[...])

    copies[5].wait()
    inc = vinc[...]
    incT = jnp.transpose(inc)
    deg = jnp.sum(incT, axis=1, keepdims=True)

    M = jax.lax.dot_general(inc, X, (((0,), (0,)), ((), ())),
                            preferred_element_type=f32)
    M = M / deg
    copies[6].wait()
    edge = jax.lax.dot_general(vw2[...], M, (((0,), (0,)), ((), ())),
                               preferred_element_type=f32)

    for c in copies[7:]:
        c.wait()
    a_full = va[...]
    s_n = jnp.dot(X, a_full[:d], preferred_element_type=f32)
    s_e = jnp.dot(edge, a_full[d:], preferred_element_type=f32)

    priE = vpri[0][:, 1:]
    a2_full = va2[...]
    t_e = _leaky(jnp.dot(edge, a2_full[:d], preferred_element_type=f32)
                 + jnp.dot(priE, a2_full[d:],
                           preferred_element_type=f32))

    pair = _leaky(s_e + jnp.transpose(s_n))

    a3b = _bf(va3[...])
    logits = _leaky(a3b[0, 0] * _bf(t_e) + a3b[1, 0] * _bf(pair))

    att = jnp.where(incT > 0, logits, f32(-9e15))
    P = jax.nn.softmax(att, axis=-1)

    node_ref[...] = jax.lax.dot_general(edge, P, (((0,), (0,)), ((), ())),
                                        preferred_element_type=f32)
    edge_ref[...] = edge


def kernel(x, m, pri_e, pri_n, hidden, incidence, weight, bias, weight2, a, a2, a3):
    B, F, N = x.shape
    E = incidence.shape[1]
    d = weight.shape[1]
    P1 = pri_e.shape[2]
    f32 = jnp.float32
    any_spec = pl.BlockSpec(memory_space=pl.ANY)

    node_dN, edge_Ed = pl.pallas_call(
        _fused_kernel,
        in_specs=[any_spec] * 11,
        out_shape=(
            jax.ShapeDtypeStruct((d, N), f32),
            jax.ShapeDtypeStruct((E, d), f32),
        ),
        scratch_shapes=[
            pltpu.VMEM((B, F, N), f32),
            pltpu.VMEM((B, F, N), f32),
            pltpu.VMEM((B, F, N), f32),
            pltpu.VMEM((B, E, P1), f32),
            pltpu.VMEM((N, E), f32),
            pltpu.VMEM((3 * F, d), f32),
            pltpu.VMEM((1, N), f32),
            pltpu.VMEM((E, E), f32),
            pltpu.VMEM((2 * d, 1), f32),
            pltpu.VMEM((2 * d, 1), f32),
            pltpu.VMEM((2, 1), f32),
            pltpu.SemaphoreType.DMA((11,)),
        ],
    )(x, m, hidden, pri_e, incidence, weight, bias.reshape(1, N),
      weight2, a, a2, a3)

    return node_dN[None], edge_Ed[None]
